# baseline (speedup 1.0000x reference)
import jax
import jax.numpy as jnp
from jax.experimental import pallas as pl

_B = 8


def _vote_body(x_ref, o_ref):
    b, l, c = x_ref.shape
    iota = jax.lax.broadcasted_iota(jnp.int32, (b, c), 1)
    big = jnp.int32(c)

    def step(j, counts):
        xl = x_ref[:, j, :]
        m = jnp.max(xl, axis=1, keepdims=True)
        idx = jnp.min(jnp.where(xl == m, iota, big), axis=1, keepdims=True)
        return counts + (iota == idx).astype(jnp.float32)

    counts = jax.lax.fori_loop(0, l, step, jnp.zeros((b, c), jnp.float32),
                               unroll=2)
    m = jnp.max(counts, axis=1, keepdims=True)
    win = jnp.min(jnp.where(counts == m, iota, big), axis=1, keepdims=True)
    o_ref[...] = (iota == win).astype(jnp.float32)


def kernel(x):
    n, l, c = x.shape
    return pl.pallas_call(
        _vote_body,
        grid=(n // _B,),
        in_specs=[pl.BlockSpec((_B, l, c), lambda i: (i, 0, 0))],
        out_specs=pl.BlockSpec((_B, c), lambda i: (i, 0)),
        out_shape=jax.ShapeDtypeStruct((n, c), jnp.float32),
    )(x)


# speedup vs baseline: 2.6359x; 2.6359x over previous
import jax
import jax.numpy as jnp
from jax.experimental import pallas as pl

_BE = 8


def _vote_body(x_ref, o_ref):
    r, c = x_ref.shape
    be = o_ref.shape[0]
    l = r // be
    xb = x_ref[...]
    iota = jax.lax.broadcasted_iota(jnp.int32, (r, c), 1)
    big = jnp.int32(c)
    m = jnp.max(xb, axis=1, keepdims=True)
    idx = jnp.min(jnp.where(xb == m, iota, big), axis=1, keepdims=True)
    votes = (iota == idx).astype(jnp.float32)
    seg = (jax.lax.broadcasted_iota(jnp.int32, (be, r), 1) // l
           == jax.lax.broadcasted_iota(jnp.int32, (be, r), 0)
           ).astype(jnp.float32)
    counts = jax.lax.dot_general(
        seg, votes, (((1,), (0,)), ((), ())),
        preferred_element_type=jnp.float32)
    iota_e = jax.lax.broadcasted_iota(jnp.int32, (be, c), 1)
    m2 = jnp.max(counts, axis=1, keepdims=True)
    win = jnp.min(jnp.where(counts == m2, iota_e, big), axis=1, keepdims=True)
    o_ref[...] = (iota_e == win).astype(jnp.float32)


def kernel(x):
    n, l, c = x.shape
    x2 = x.reshape(n * l, c)
    return pl.pallas_call(
        _vote_body,
        grid=(n // _BE,),
        in_specs=[pl.BlockSpec((_BE * l, c), lambda i: (i, 0))],
        out_specs=pl.BlockSpec((_BE, c), lambda i: (i, 0)),
        out_shape=jax.ShapeDtypeStruct((n, c), jnp.float32),
    )(x2)


# speedup vs baseline: 3.4487x; 1.3084x over previous
import jax
import jax.numpy as jnp
from jax.experimental import pallas as pl

_BE = 8


def _vote_body(x_ref, o_ref):
    be, l, c = x_ref.shape
    r = be * l
    xb = x_ref[...].reshape(r, c)
    iota = jax.lax.broadcasted_iota(jnp.int32, (r, c), 1)
    big = jnp.int32(c)
    m = jnp.max(xb, axis=1, keepdims=True)
    idx = jnp.min(jnp.where(xb == m, iota, big), axis=1, keepdims=True)
    votes = (iota == idx).astype(jnp.float32)
    seg = (jax.lax.broadcasted_iota(jnp.int32, (be, r), 1) // l
           == jax.lax.broadcasted_iota(jnp.int32, (be, r), 0)
           ).astype(jnp.float32)
    counts = jax.lax.dot_general(
        seg, votes, (((1,), (0,)), ((), ())),
        preferred_element_type=jnp.float32)
    iota_e = jax.lax.broadcasted_iota(jnp.int32, (be, c), 1)
    m2 = jnp.max(counts, axis=1, keepdims=True)
    win = jnp.min(jnp.where(counts == m2, iota_e, big), axis=1, keepdims=True)
    o_ref[...] = (iota_e == win).astype(jnp.float32)


def kernel(x):
    n, l, c = x.shape
    return pl.pallas_call(
        _vote_body,
        grid=(n // _BE,),
        in_specs=[pl.BlockSpec((_BE, l, c), lambda i: (i, 0, 0))],
        out_specs=pl.BlockSpec((_BE, c), lambda i: (i, 0)),
        out_shape=jax.ShapeDtypeStruct((n, c), jnp.float32),
    )(x)


# speedup vs baseline: 3.8089x; 1.1044x over previous
import jax
import jax.numpy as jnp
from jax.experimental import pallas as pl

_BE = 32


def _vote_body(x_ref, o_ref):
    be, l, c = x_ref.shape
    r = be * l
    xb = x_ref[...].reshape(r, c)
    iota = jax.lax.broadcasted_iota(jnp.int32, (r, c), 1)
    big = jnp.int32(c)
    m = jnp.max(xb, axis=1, keepdims=True)
    idx = jnp.min(jnp.where(xb == m, iota, big), axis=1, keepdims=True)
    votes = (iota == idx).astype(jnp.float32)
    seg = (jax.lax.broadcasted_iota(jnp.int32, (be, r), 1) // l
           == jax.lax.broadcasted_iota(jnp.int32, (be, r), 0)
           ).astype(jnp.float32)
    counts = jax.lax.dot_general(
        seg, votes, (((1,), (0,)), ((), ())),
        preferred_element_type=jnp.float32)
    iota_e = jax.lax.broadcasted_iota(jnp.int32, (be, c), 1)
    m2 = jnp.max(counts, axis=1, keepdims=True)
    win = jnp.min(jnp.where(counts == m2, iota_e, big), axis=1, keepdims=True)
    o_ref[...] = (iota_e == win).astype(jnp.float32)


def kernel(x):
    n, l, c = x.shape
    return pl.pallas_call(
        _vote_body,
        grid=(n // _BE,),
        in_specs=[pl.BlockSpec((_BE, l, c), lambda i: (i, 0, 0))],
        out_specs=pl.BlockSpec((_BE, c), lambda i: (i, 0)),
        out_shape=jax.ShapeDtypeStruct((n, c), jnp.float32),
    )(x)


# speedup vs baseline: 3.8697x; 1.0160x over previous
import jax
import jax.numpy as jnp
from jax.experimental import pallas as pl

_BE = 8
_NS = 4


def _vote_body(*refs):
    x_refs = refs[:-1]
    o_ref = refs[-1]
    be, l, c = x_refs[0].shape
    r = be * l
    iota = jax.lax.broadcasted_iota(jnp.int32, (r, c), 1)
    big = jnp.int32(c)
    seg = (jax.lax.broadcasted_iota(jnp.int32, (be, r), 1) // l
           == jax.lax.broadcasted_iota(jnp.int32, (be, r), 0)
           ).astype(jnp.float32)
    iota_e = jax.lax.broadcasted_iota(jnp.int32, (be, c), 1)
    for s, x_ref in enumerate(x_refs):
        xb = x_ref[...].reshape(r, c)
        m = jnp.max(xb, axis=1, keepdims=True)
        idx = jnp.min(jnp.where(xb == m, iota, big), axis=1, keepdims=True)
        votes = (iota == idx).astype(jnp.float32)
        counts = jax.lax.dot_general(
            seg, votes, (((1,), (0,)), ((), ())),
            preferred_element_type=jnp.float32)
        m2 = jnp.max(counts, axis=1, keepdims=True)
        win = jnp.min(jnp.where(counts == m2, iota_e, big), axis=1,
                      keepdims=True)
        o_ref[s * be:(s + 1) * be, :] = (iota_e == win).astype(jnp.float32)


def kernel(x):
    n, l, c = x.shape
    in_specs = [
        pl.BlockSpec((_BE, l, c), lambda i, s=s: (i * _NS + s, 0, 0))
        for s in range(_NS)
    ]
    return pl.pallas_call(
        _vote_body,
        grid=(n // (_BE * _NS),),
        in_specs=in_specs,
        out_specs=pl.BlockSpec((_BE * _NS, c), lambda i: (i, 0)),
        out_shape=jax.ShapeDtypeStruct((n, c), jnp.float32),
    )(*([x] * _NS))


# speedup vs baseline: 9.8788x; 2.5528x over previous
import jax
import jax.numpy as jnp
from jax.experimental import pallas as pl

_LB = 2
_BE = 16


def _argmax_body(x_ref, o_ref):
    lb, c, n = x_ref.shape
    row_iota = jax.lax.broadcasted_iota(jnp.int32, (c, n), 0)
    big = jnp.int32(c)
    for j in range(lb):
        x2 = x_ref[j]
        m = jnp.max(x2, axis=0, keepdims=True)
        idx = jnp.min(jnp.where(x2 == m, row_iota, big), axis=0,
                      keepdims=True)
        o_ref[j] = idx


def _vote_body(i_ref, o_ref):
    r, one = i_ref.shape
    be, c = o_ref.shape
    l = r // be
    idx = i_ref[...]
    iota = jax.lax.broadcasted_iota(jnp.int32, (r, c), 1)
    votes = (iota == idx).astype(jnp.float32)
    seg = (jax.lax.broadcasted_iota(jnp.int32, (be, r), 1) // l
           == jax.lax.broadcasted_iota(jnp.int32, (be, r), 0)
           ).astype(jnp.float32)
    counts = jax.lax.dot_general(
        seg, votes, (((1,), (0,)), ((), ())),
        preferred_element_type=jnp.float32)
    iota_e = jax.lax.broadcasted_iota(jnp.int32, (be, c), 1)
    big = jnp.int32(c)
    m2 = jnp.max(counts, axis=1, keepdims=True)
    win = jnp.min(jnp.where(counts == m2, iota_e, big), axis=1,
                  keepdims=True)
    o_ref[...] = (iota_e == win).astype(jnp.float32)


def kernel(x):
    n, l, c = x.shape
    xt = jnp.transpose(x, (1, 2, 0))
    idx = pl.pallas_call(
        _argmax_body,
        grid=(l // _LB,),
        in_specs=[pl.BlockSpec((_LB, c, n), lambda i: (i, 0, 0))],
        out_specs=pl.BlockSpec((_LB, 1, n), lambda i: (i, 0, 0)),
        out_shape=jax.ShapeDtypeStruct((l, 1, n), jnp.int32),
    )(xt)
    idx_t = jnp.transpose(idx.reshape(l, n), (1, 0)).reshape(n * l, 1)
    return pl.pallas_call(
        _vote_body,
        grid=(n // _BE,),
        in_specs=[pl.BlockSpec((_BE * l, 1), lambda i: (i, 0))],
        out_specs=pl.BlockSpec((_BE, c), lambda i: (i, 0)),
        out_shape=jax.ShapeDtypeStruct((n, c), jnp.float32),
    )(idx_t)
